# baseline (speedup 1.0000x reference)
import functools
import math

import jax
import jax.numpy as jnp
from jax import lax
from jax.experimental import pallas as pl
from jax.experimental.pallas import tpu as pltpu
from jax.experimental.pallas import tpu_sc as plsc

_info = plsc.get_sparse_core_info()
_NC, _NS, _L = _info.num_cores, _info.num_subcores, _info.num_lanes
_NW = _NC * _NS


def _make_emb_kernel(S0, S1, V, D, b_per_w, C, NBUF):
    nch = b_per_w // C
    scale = math.sqrt(D)
    w_per_row = S1 // b_per_w
    mesh = plsc.VectorSubcoreMesh(core_axis_name="c", subcore_axis_name="s")

    @functools.partial(
        pl.kernel,
        out_type=jax.ShapeDtypeStruct((S0, S1, D), jnp.float32),
        mesh=mesh,
        scratch_types=[
            pltpu.VMEM((b_per_w,), jnp.int32),
        ]
        + [pltpu.VMEM((C, D), jnp.float32) for _ in range(NBUF)]
        + [pltpu.SemaphoreType.DMA for _ in range(2 * NBUF)],
    )
    def emb(idx_hbm, lut_hbm, out_hbm, idx_v, *scratch):
        bufs = scratch[:NBUF]
        gsems = scratch[NBUF : 2 * NBUF]
        ssems = scratch[2 * NBUF :]
        wid = lax.axis_index("s") * _NC + lax.axis_index("c")
        plane = wid // w_per_row
        col0 = (wid % w_per_row) * b_per_w
        pltpu.sync_copy(idx_hbm.at[plane, pl.ds(col0, b_per_w)], idx_v)

        def scale_buf(buf):
            def row_body(r, _):
                for j in range(D // _L):
                    sl = pl.ds(j * _L, _L)
                    buf[r, sl] = buf[r, sl] * scale
                return _

            lax.fori_loop(0, C, row_body, 0, unroll=2)

        def gather_chunk(c, b):
            return pltpu.make_async_copy(
                lut_hbm.at[idx_v.at[pl.ds(c * C, C)]], bufs[b], gsems[b]
            )

        def scatter_chunk(c, b):
            return pltpu.make_async_copy(
                bufs[b], out_hbm.at[plane, pl.ds(col0 + c * C, C)], ssems[b]
            )

        for c in range(NBUF):
            gather_chunk(c, c).start()

        def group_body(i, _):
            for k in range(NBUF):
                c = i * NBUF + k
                gather_chunk(c, k).wait()
                e = c + NBUF - 1

                @pl.when(jnp.logical_and(c >= 1, e < nch))
                def _refill_prev_buffer():
                    scatter_chunk(c - 1, (k - 1) % NBUF).wait()
                    gather_chunk(e, (k - 1) % NBUF).start()

                scale_buf(bufs[k])
                scatter_chunk(c, k).start()
            return _

        lax.fori_loop(0, nch // NBUF, group_body, 0, unroll=False)
        for c in range(nch - NBUF, nch):
            scatter_chunk(c, c % NBUF).wait()

    return emb


@jax.jit
def kernel(X, lut):
    S0, S1 = X.shape
    V, D = lut.shape
    B = S0 * S1
    b_per_w = B // _NW
    C = 32
    NBUF = 4
    emb = _make_emb_kernel(S0, S1, V, D, b_per_w, C, NBUF)
    return emb(X.astype(jnp.int32), lut)


# speedup vs baseline: 1.3472x; 1.3472x over previous
import functools
import math

import jax
import jax.numpy as jnp
from jax import lax
from jax.experimental import pallas as pl
from jax.experimental.pallas import tpu as pltpu
from jax.experimental.pallas import tpu_sc as plsc

_info = plsc.get_sparse_core_info()
_NC, _NS, _L = _info.num_cores, _info.num_subcores, _info.num_lanes
_NW = _NC * _NS


def _make_emb_kernel(S0, S1, V, D, b_per_w, C, NBUF):
    nch = b_per_w // C
    scale = math.sqrt(D)
    w_per_row = S1 // b_per_w
    mesh = plsc.VectorSubcoreMesh(core_axis_name="c", subcore_axis_name="s")

    @functools.partial(
        pl.kernel,
        out_type=jax.ShapeDtypeStruct((S0, S1, D), jnp.float32),
        mesh=mesh,
        scratch_types=[
            pltpu.VMEM((b_per_w,), jnp.int32),
        ]
        + [pltpu.VMEM((C, D), jnp.float32) for _ in range(NBUF)]
        + [pltpu.SemaphoreType.DMA for _ in range(2 * NBUF)],
    )
    def emb(idx_hbm, lut_hbm, out_hbm, idx_v, *scratch):
        bufs = scratch[:NBUF]
        gsems = scratch[NBUF : 2 * NBUF]
        ssems = scratch[2 * NBUF :]
        wid = lax.axis_index("s") * _NC + lax.axis_index("c")
        plane = wid // w_per_row
        col0 = (wid % w_per_row) * b_per_w
        pltpu.sync_copy(idx_hbm.at[plane, pl.ds(col0, b_per_w)], idx_v)

        def scale_buf(buf):
            @plsc.parallel_loop(0, C)
            def _rows(r):
                for j in range(D // _L):
                    sl = pl.ds(j * _L, _L)
                    buf[r, sl] = buf[r, sl] * scale

        def gather_chunk(c, b):
            return pltpu.make_async_copy(
                lut_hbm.at[idx_v.at[pl.ds(c * C, C)]], bufs[b], gsems[b]
            )

        def scatter_chunk(c, b):
            return pltpu.make_async_copy(
                bufs[b], out_hbm.at[plane, pl.ds(col0 + c * C, C)], ssems[b]
            )

        for c in range(NBUF):
            gather_chunk(c, c).start()

        def group_body(i, _):
            for k in range(NBUF):
                c = i * NBUF + k
                gather_chunk(c, k).wait()
                e = c + NBUF - 1

                @pl.when(jnp.logical_and(c >= 1, e < nch))
                def _refill_prev_buffer():
                    scatter_chunk(c - 1, (k - 1) % NBUF).wait()
                    gather_chunk(e, (k - 1) % NBUF).start()

                scale_buf(bufs[k])
                scatter_chunk(c, k).start()
            return _

        lax.fori_loop(0, nch // NBUF, group_body, 0, unroll=False)
        for c in range(nch - NBUF, nch):
            scatter_chunk(c, c % NBUF).wait()

    return emb


@jax.jit
def kernel(X, lut):
    S0, S1 = X.shape
    V, D = lut.shape
    B = S0 * S1
    b_per_w = B // _NW
    C = 32
    NBUF = 4
    emb = _make_emb_kernel(S0, S1, V, D, b_per_w, C, NBUF)
    return emb(X.astype(jnp.int32), lut)


# speedup vs baseline: 1.3663x; 1.0142x over previous
import functools
import math

import jax
import jax.numpy as jnp
from jax import lax
from jax.experimental import pallas as pl
from jax.experimental.pallas import tpu as pltpu
from jax.experimental.pallas import tpu_sc as plsc

_info = plsc.get_sparse_core_info()
_NC, _NS, _L = _info.num_cores, _info.num_subcores, _info.num_lanes
_NW = _NC * _NS


def _make_emb_kernel(S0, S1, V, D, b_per_w, C, NBUF):
    nch = b_per_w // C
    scale = math.sqrt(D)
    w_per_row = S1 // b_per_w
    mesh = plsc.VectorSubcoreMesh(core_axis_name="c", subcore_axis_name="s")

    @functools.partial(
        pl.kernel,
        out_type=jax.ShapeDtypeStruct((S0, S1, D), jnp.float32),
        mesh=mesh,
        scratch_types=[
            pltpu.VMEM((b_per_w,), jnp.int32),
        ]
        + [pltpu.VMEM((C, D), jnp.float32) for _ in range(NBUF)]
        + [pltpu.SemaphoreType.DMA for _ in range(2 * NBUF)],
    )
    def emb(idx_hbm, lut_hbm, out_hbm, idx_v, *scratch):
        bufs = scratch[:NBUF]
        gsems = scratch[NBUF : 2 * NBUF]
        ssems = scratch[2 * NBUF :]
        wid = lax.axis_index("s") * _NC + lax.axis_index("c")
        plane = wid // w_per_row
        col0 = (wid % w_per_row) * b_per_w
        pltpu.sync_copy(idx_hbm.at[plane, pl.ds(col0, b_per_w)], idx_v)

        def scale_buf(buf):
            def row_body(r, _):
                for j in range(D // _L):
                    sl = pl.ds(j * _L, _L)
                    buf[r, sl] = buf[r, sl] * scale
                return _

            lax.fori_loop(0, C, row_body, 0, unroll=False)

        def gather_chunk(c, b):
            return pltpu.make_async_copy(
                lut_hbm.at[idx_v.at[pl.ds(c * C, C)]], bufs[b], gsems[b]
            )

        def scatter_chunk(c, b):
            return pltpu.make_async_copy(
                bufs[b], out_hbm.at[plane, pl.ds(col0 + c * C, C)], ssems[b]
            )

        for c in range(NBUF):
            gather_chunk(c, c).start()

        def group_body(i, _):
            for k in range(NBUF):
                c = i * NBUF + k
                gather_chunk(c, k).wait()
                scale_buf(bufs[k])
                scatter_chunk(c, k).start()
                e = c + NBUF - 1

                @pl.when(jnp.logical_and(c >= 1, e < nch))
                def _refill_prev_buffer():
                    scatter_chunk(c - 1, (k - 1) % NBUF).wait()
                    gather_chunk(e, (k - 1) % NBUF).start()
            return _

        lax.fori_loop(0, nch // NBUF, group_body, 0, unroll=False)
        for c in range(nch - NBUF, nch):
            scatter_chunk(c, c % NBUF).wait()

    return emb


@jax.jit
def kernel(X, lut):
    S0, S1 = X.shape
    V, D = lut.shape
    B = S0 * S1
    b_per_w = B // _NW
    C = 32
    NBUF = 4
    emb = _make_emb_kernel(S0, S1, V, D, b_per_w, C, NBUF)
    return emb(X.astype(jnp.int32), lut)
